# Initial kernel scaffold; baseline (speedup 1.0000x reference)
import functools

import jax
import jax.numpy as jnp
from jax.experimental import pallas as pl
from jax.experimental.pallas import tpu as pltpu

NUM_EXPERTS = 8
TOP_K = 2
D_MODEL = 1024
D_FF = 2048


def _router_body(x_ref, rw_ref, w8_ref):
    x = x_ref[...]
    rw = rw_ref[...]
    logits = jax.lax.dot_general(
        x, rw, (((1,), (1,)), ((), ())),
        preferred_element_type=jnp.float32,
        precision=jax.lax.Precision.HIGHEST,
    )
    m = jnp.max(logits, axis=-1, keepdims=True)
    e = jnp.exp(logits - m)
    probs = e / jnp.sum(e, axis=-1, keepdims=True)
    eidx = jax.lax.broadcasted_iota(jnp.int32, probs.shape, 1)
    p1 = jnp.max(probs, axis=-1, keepdims=True)
    a1 = jnp.min(jnp.where(probs == p1, eidx, NUM_EXPERTS), axis=-1, keepdims=True)
    probs2 = jnp.where(eidx == a1, -1.0, probs)
    p2 = jnp.max(probs2, axis=-1, keepdims=True)
    a2 = jnp.min(jnp.where(probs2 == p2, eidx, NUM_EXPERTS), axis=-1, keepdims=True)
    s = p1 + p2
    w8 = jnp.where(eidx == a1, p1 / s, 0.0) + jnp.where(eidx == a2, p2 / s, 0.0)
    w8_ref[...] = w8.astype(jnp.float32)


def _moe_body(x_ref, wg_ref, wu_ref, wd_ref, w8_ref, out_ref, *, blk_t):
    e = pl.program_id(0)
    i = pl.program_id(1)
    xb = x_ref[...].astype(jnp.bfloat16)
    wg = wg_ref[0].astype(jnp.bfloat16)
    wu = wu_ref[0].astype(jnp.bfloat16)
    wd = wd_ref[0].astype(jnp.bfloat16)
    g = jax.lax.dot_general(xb, wg, (((1,), (1,)), ((), ())),
                            preferred_element_type=jnp.float32)
    u = jax.lax.dot_general(xb, wu, (((1,), (1,)), ((), ())),
                            preferred_element_type=jnp.float32)
    h = (g / (1.0 + jnp.exp(-g))) * u
    y = jax.lax.dot_general(h.astype(jnp.bfloat16), wd, (((1,), (1,)), ((), ())),
                            preferred_element_type=jnp.float32)
    w = w8_ref[:, e][:, None]
    y = y * w
    sl = pl.ds(i * blk_t, blk_t)

    @pl.when(e == 0)
    def _init():
        out_ref[sl, :] = y

    @pl.when(e != 0)
    def _acc():
        out_ref[sl, :] += y


def kernel(hidden_states, router_w, w_gate, w_up, w_down):
    b, s, d = hidden_states.shape
    T = b * s
    x = hidden_states.reshape(T, d)

    w8 = pl.pallas_call(
        _router_body,
        out_shape=jax.ShapeDtypeStruct((T, NUM_EXPERTS), jnp.float32),
    )(x, router_w)

    BLK_T = 512
    n_tb = T // BLK_T
    out = pl.pallas_call(
        functools.partial(_moe_body, blk_t=BLK_T),
        grid=(NUM_EXPERTS, n_tb),
        in_specs=[
            pl.BlockSpec((BLK_T, D_MODEL), lambda e, i: (i, 0)),
            pl.BlockSpec((1, D_FF, D_MODEL), lambda e, i: (e, 0, 0)),
            pl.BlockSpec((1, D_FF, D_MODEL), lambda e, i: (e, 0, 0)),
            pl.BlockSpec((1, D_MODEL, D_FF), lambda e, i: (e, 0, 0)),
            pl.BlockSpec((BLK_T, NUM_EXPERTS), lambda e, i: (i, 0)),
        ],
        out_specs=pl.BlockSpec((T, D_MODEL), lambda e, i: (0, 0)),
        out_shape=jax.ShapeDtypeStruct((T, D_MODEL), jnp.float32),
    )(x, w_gate, w_up, w_down, w8)

    return out.reshape(b, s, D_MODEL)


# speedup vs baseline: 1.4046x; 1.4046x over previous
import functools

import jax
import jax.numpy as jnp
from jax.experimental import pallas as pl
from jax.experimental.pallas import tpu as pltpu

NUM_EXPERTS = 8
TOP_K = 2
D_MODEL = 1024
D_FF = 2048


def _router_body(x_ref, rw_ref, w8_ref):
    x = x_ref[...]
    rw = rw_ref[...]
    logits = jax.lax.dot_general(
        x, rw, (((1,), (1,)), ((), ())),
        preferred_element_type=jnp.float32,
    )
    m = jnp.max(logits, axis=-1, keepdims=True)
    e = jnp.exp(logits - m)
    probs = e / jnp.sum(e, axis=-1, keepdims=True)
    eidx = jax.lax.broadcasted_iota(jnp.int32, probs.shape, 1)
    p1 = jnp.max(probs, axis=-1, keepdims=True)
    a1 = jnp.min(jnp.where(probs == p1, eidx, NUM_EXPERTS), axis=-1, keepdims=True)
    probs2 = jnp.where(eidx == a1, -1.0, probs)
    p2 = jnp.max(probs2, axis=-1, keepdims=True)
    a2 = jnp.min(jnp.where(probs2 == p2, eidx, NUM_EXPERTS), axis=-1, keepdims=True)
    s = p1 + p2
    w8 = jnp.where(eidx == a1, p1 / s, 0.0) + jnp.where(eidx == a2, p2 / s, 0.0)
    w8_ref[...] = w8.astype(jnp.float32)


def _moe_body(x_ref, wg_ref, wu_ref, wd_ref, w8_ref, out_ref, *, blk_t):
    e = pl.program_id(0)
    f = pl.program_id(1)
    i = pl.program_id(2)
    sl = pl.ds(i * blk_t, blk_t)
    xb = x_ref[sl, :].astype(jnp.bfloat16)
    wg = wg_ref[0].astype(jnp.bfloat16)
    wu = wu_ref[0].astype(jnp.bfloat16)
    wd = wd_ref[0].astype(jnp.bfloat16)
    g = jax.lax.dot_general(xb, wg, (((1,), (1,)), ((), ())),
                            preferred_element_type=jnp.float32)
    u = jax.lax.dot_general(xb, wu, (((1,), (1,)), ((), ())),
                            preferred_element_type=jnp.float32)
    h = (g / (1.0 + jnp.exp(-g))) * u
    y = jax.lax.dot_general(h.astype(jnp.bfloat16), wd, (((1,), (1,)), ((), ())),
                            preferred_element_type=jnp.float32)
    w8 = w8_ref[sl, :]
    lane = jax.lax.broadcasted_iota(jnp.int32, w8.shape, 1)
    w = jnp.sum(jnp.where(lane == e, w8, 0.0), axis=-1, keepdims=True)
    y = y * w

    @pl.when((e == 0) & (f == 0))
    def _init():
        out_ref[sl, :] = y

    @pl.when((e != 0) | (f != 0))
    def _acc():
        out_ref[sl, :] += y


def kernel(hidden_states, router_w, w_gate, w_up, w_down):
    b, s, d = hidden_states.shape
    T = b * s
    x = hidden_states.reshape(T, d)

    w8 = pl.pallas_call(
        _router_body,
        out_shape=jax.ShapeDtypeStruct((T, NUM_EXPERTS), jnp.float32),
    )(x, router_w)

    BLK_T = 512
    BLK_F = 1024
    n_tb = T // BLK_T
    n_fb = D_FF // BLK_F
    out = pl.pallas_call(
        functools.partial(_moe_body, blk_t=BLK_T),
        grid=(NUM_EXPERTS, n_fb, n_tb),
        in_specs=[
            pl.BlockSpec((T, D_MODEL), lambda e, f, i: (0, 0)),
            pl.BlockSpec((1, BLK_F, D_MODEL), lambda e, f, i: (e, f, 0)),
            pl.BlockSpec((1, BLK_F, D_MODEL), lambda e, f, i: (e, f, 0)),
            pl.BlockSpec((1, D_MODEL, BLK_F), lambda e, f, i: (e, 0, f)),
            pl.BlockSpec((T, NUM_EXPERTS), lambda e, f, i: (0, 0)),
        ],
        out_specs=pl.BlockSpec((T, D_MODEL), lambda e, f, i: (0, 0)),
        out_shape=jax.ShapeDtypeStruct((T, D_MODEL), jnp.float32),
    )(x, w_gate, w_up, w_down, w8)

    return out.reshape(b, s, D_MODEL)
